# baseline (speedup 1.0000x reference)
import jax
import jax.numpy as jnp
from jax import lax
from jax.experimental import pallas as pl
from jax.experimental.pallas import tpu as pltpu

_B, _C, _H, _W = 64, 768, 14, 14
_S = _H * _W
_E = 64
_RPB = 32
_NR = _C // _RPB
_K = _RPB * _S


def _body(x_ref, wt_ref, b_ref, o_ref, g_ref, acc_ref):
    @pl.when(pl.program_id(0) == 0)
    def _init():
        col = lax.broadcasted_iota(jnp.int32, (_K, _RPB), 0)
        g = lax.broadcasted_iota(jnp.int32, (_K, _RPB), 1)
        lo = g * _S
        g_ref[...] = jnp.where((col >= lo) & (col < lo + _S), 1.0, 0.0)
        acc_ref[...] = jnp.zeros((_B, _E), jnp.float32)

    m = x_ref[...].reshape(_B, _K)
    p = lax.dot_general(
        m, g_ref[...], (((1,), (0,)), ((), ())),
        preferred_element_type=jnp.float32,
    )
    acc_ref[...] += lax.dot_general(
        p, wt_ref[0], (((1,), (0,)), ((), ())),
        preferred_element_type=jnp.float32,
    )

    @pl.when(pl.program_id(0) == _NR - 1)
    def _finish():
        logits = acc_ref[...] * (1.0 / _S) + b_ref[...]
        mx = jnp.max(logits, axis=1, keepdims=True)
        e = jnp.exp(logits - mx)
        o_ref[...] = e / jnp.sum(e, axis=1, keepdims=True)


def kernel(x, W, b):
    x4 = x.reshape(_B, _NR, 1, _K)
    b2 = b.reshape(1, _E)
    wt = W.T.reshape(_NR, _RPB, _E)
    return pl.pallas_call(
        _body,
        grid=(_NR,),
        in_specs=[
            pl.BlockSpec((_B, 1, 1, _K), lambda i: (0, i, 0, 0)),
            pl.BlockSpec((1, _RPB, _E), lambda i: (i, 0, 0)),
            pl.BlockSpec((1, _E), lambda i: (0, 0)),
        ],
        out_specs=pl.BlockSpec((_B, _E), lambda i: (0, 0)),
        out_shape=jax.ShapeDtypeStruct((_B, _E), jnp.float32),
        scratch_shapes=[
            pltpu.VMEM((_K, _RPB), jnp.float32),
            pltpu.VMEM((_B, _E), jnp.float32),
        ],
    )(x4, wt, b2)


# speedup vs baseline: 14.3503x; 14.3503x over previous
import jax
import jax.numpy as jnp
from jax import lax
from jax.experimental import pallas as pl
from jax.experimental.pallas import tpu as pltpu

_B, _C, _H, _W = 64, 768, 14, 14
_S = _H * _W
_E = 64
_PC = 14
_NSTEP = _S // _PC


def _body(x_ref, wt_ref, b_ref, o_ref, acc_ref):
    part = jnp.sum(x_ref[...], axis=0)

    @pl.when(pl.program_id(0) == 0)
    def _init():
        acc_ref[...] = part

    @pl.when(pl.program_id(0) > 0)
    def _accum():
        acc_ref[...] += part

    @pl.when(pl.program_id(0) == _NSTEP - 1)
    def _finish():
        pooled = acc_ref[...] * (1.0 / _S)
        logits = lax.dot_general(
            pooled, wt_ref[...], (((1,), (0,)), ((), ())),
            preferred_element_type=jnp.float32,
        ) + b_ref[...]
        mx = jnp.max(logits, axis=1, keepdims=True)
        e = jnp.exp(logits - mx)
        o_ref[...] = e / jnp.sum(e, axis=1, keepdims=True)


def kernel(x, W, b):
    xp = jnp.transpose(x, (2, 3, 0, 1)).reshape(_S, _B, _C)
    b2 = b.reshape(1, _E)
    wt = W.T
    return pl.pallas_call(
        _body,
        grid=(_NSTEP,),
        in_specs=[
            pl.BlockSpec((_PC, _B, _C), lambda i: (i, 0, 0)),
            pl.BlockSpec((_C, _E), lambda i: (0, 0)),
            pl.BlockSpec((1, _E), lambda i: (0, 0)),
        ],
        out_specs=pl.BlockSpec((_B, _E), lambda i: (0, 0)),
        out_shape=jax.ShapeDtypeStruct((_B, _E), jnp.float32),
        scratch_shapes=[pltpu.VMEM((_B, _C), jnp.float32)],
    )(xp, wt, b2)
